# Initial kernel scaffold; baseline (speedup 1.0000x reference)
import jax
import jax.numpy as jnp
from jax.experimental import pallas as pl


def kernel(x, edge_index, edge_type, edge_attr, node2graph, params):
    raise NotImplementedError("write your pallas kernel here")


# speedup vs baseline: 1.6088x; 1.6088x over previous
import functools
from functools import partial

import jax
import jax.numpy as jnp
import numpy as np
from jax.experimental import pallas as pl
from jax.experimental.pallas import tpu as pltpu

N_NODES = 10000
N_EDGES = 320000
NUM_REL = 4
EDGE_DIM = 16
NUM_GRAPHS = 64
EPS = 1e-10
BN_SCALE = 1.0 / np.sqrt(1.0 + 1e-5)

ROW_BLK = 2000


def _zr_body(n_pieces, *refs):
    piece_refs = refs[:n_pieces]
    wz_ref, wr_ref, b_ref, z_ref, r_ref = refs[n_pieces:]
    if n_pieces == 1:
        a = piece_refs[0][...]
    else:
        a = jnp.concatenate([p[...] for p in piece_refs], axis=1)
    z_ref[...] = jnp.dot(a, wz_ref[...], preferred_element_type=jnp.float32)
    r_ref[...] = jnp.dot(a, wr_ref[...], preferred_element_type=jnp.float32) + b_ref[...]


def _matmul_zr(pieces, Wz, Wr, b):
    n = pieces[0].shape[0]
    zout = Wz.shape[1]
    rout = Wr.shape[1]
    grid = n // ROW_BLK
    in_specs = [pl.BlockSpec((ROW_BLK, p.shape[1]), lambda i: (i, 0)) for p in pieces]
    in_specs += [
        pl.BlockSpec(Wz.shape, lambda i: (0, 0)),
        pl.BlockSpec(Wr.shape, lambda i: (0, 0)),
        pl.BlockSpec((1, rout), lambda i: (0, 0)),
    ]
    out_specs = [
        pl.BlockSpec((ROW_BLK, zout), lambda i: (i, 0)),
        pl.BlockSpec((ROW_BLK, rout), lambda i: (i, 0)),
    ]
    return pl.pallas_call(
        partial(_zr_body, len(pieces)),
        grid=(grid,),
        in_specs=in_specs,
        out_specs=out_specs,
        out_shape=[
            jax.ShapeDtypeStruct((n, zout), jnp.float32),
            jax.ShapeDtypeStruct((n, rout), jnp.float32),
        ],
    )(*pieces, Wz, Wr, b.reshape(1, rout))


def _cls_body(n_pieces, *refs):
    i = pl.program_id(0)
    piece_refs = refs[:n_pieces]
    w_ref, b_ref, n2g_ref, nf_ref, gs_ref, cnt_ref = refs[n_pieces:]
    if n_pieces == 1:
        a = piece_refs[0][...]
    else:
        a = jnp.concatenate([p[...] for p in piece_refs], axis=1)
    nf = jnp.dot(a, w_ref[...], preferred_element_type=jnp.float32) + b_ref[...]
    nf_ref[...] = nf
    n2g = n2g_ref[0, 0, :]
    onehot = (n2g[None, :] == jax.lax.broadcasted_iota(jnp.int32, (NUM_GRAPHS, ROW_BLK), 0)
              ).astype(jnp.float32)
    part_s = jnp.dot(onehot, nf, preferred_element_type=jnp.float32)
    part_c = jnp.sum(onehot, axis=1, keepdims=True)

    @pl.when(i == 0)
    def _():
        gs_ref[...] = jnp.zeros_like(gs_ref)
        cnt_ref[...] = jnp.zeros_like(cnt_ref)

    gs_ref[...] += part_s
    cnt_ref[...] += part_c


def _cls_readout(pieces, W, b, node2graph):
    n = pieces[0].shape[0]
    out = W.shape[1]
    grid = n // ROW_BLK
    in_specs = [pl.BlockSpec((ROW_BLK, p.shape[1]), lambda i: (i, 0)) for p in pieces]
    in_specs += [
        pl.BlockSpec(W.shape, lambda i: (0, 0)),
        pl.BlockSpec((1, out), lambda i: (0, 0)),
        pl.BlockSpec((1, 1, ROW_BLK), lambda i: (i, 0, 0)),
    ]
    out_specs = [
        pl.BlockSpec((ROW_BLK, out), lambda i: (i, 0)),
        pl.BlockSpec((NUM_GRAPHS, out), lambda i: (0, 0)),
        pl.BlockSpec((NUM_GRAPHS, 1), lambda i: (0, 0)),
    ]
    nf, gs, cnt = pl.pallas_call(
        partial(_cls_body, len(pieces)),
        grid=(grid,),
        in_specs=in_specs,
        out_specs=out_specs,
        out_shape=[
            jax.ShapeDtypeStruct((n, out), jnp.float32),
            jax.ShapeDtypeStruct((NUM_GRAPHS, out), jnp.float32),
            jax.ShapeDtypeStruct((NUM_GRAPHS, 1), jnp.float32),
        ],
    )(*pieces, W, b.reshape(1, out),
      node2graph.reshape(n // ROW_BLK, 1, ROW_BLK).astype(jnp.int32))
    gf = gs / jnp.maximum(cnt, 1.0)
    return gf, nf


def _precompute_structure(edge_index, edge_type, edge_attr, n):
    src = edge_index[0]
    dst = edge_index[1]
    seg = dst * NUM_REL + edge_type
    den = jax.ops.segment_sum(jnp.ones((N_EDGES,), jnp.float32), seg,
                              num_segments=n * NUM_REL)
    inv = 1.0 / (den + EPS)
    Ef = jax.ops.segment_sum(edge_attr, seg, num_segments=n * NUM_REL)
    F = Ef * inv[:, None]
    c = den * inv
    Phi = jnp.concatenate([F.reshape(n, NUM_REL * EDGE_DIM),
                           c.reshape(n, NUM_REL)], axis=1)
    gidx = src * NUM_REL + edge_type
    return seg, inv, Phi, gidx


def _layer_weights(p, in_dim):
    out_dim = p['W_loop'].shape[1]
    g = p['gamma'] * BN_SCALE
    Wlin = p['W_lin'].reshape(NUM_REL, in_dim, out_dim) * g
    Wz = Wlin.transpose(1, 0, 2).reshape(in_dim, NUM_REL * out_dim)
    Wz = jnp.concatenate([Wz, jnp.zeros((68, NUM_REL * out_dim), jnp.float32)], axis=0)
    U = jnp.einsum('ed,rdo->reo', p['W_edge'], Wlin)
    V = jnp.einsum('d,rdo->ro', p['b_edge'], Wlin)
    G = jnp.concatenate([U.reshape(NUM_REL * EDGE_DIM, out_dim), V], axis=0)
    Wr = jnp.concatenate([p['W_loop'] * g, G], axis=0)
    b = (p['b_lin'] + p['b_loop']) * g + p['beta']
    return Wz, Wr, b


def _edge_op(Z, R, seg, inv, gidx, n):
    out_dim = R.shape[1]
    Zv = Z.reshape(n * NUM_REL, out_dim)
    rows = Zv[gidx]
    acc = jax.ops.segment_sum(rows, seg, num_segments=n * NUM_REL)
    M = (acc * inv[:, None]).reshape(n, NUM_REL, out_dim).sum(axis=1)
    return jax.nn.relu(M + R)


def kernel(x, edge_index, edge_type, edge_attr, node2graph, params):
    n = x.shape[0]
    seg, inv, Phi, gidx = _precompute_structure(edge_index, edge_type, edge_attr, n)

    def run(pieces, p):
        in_dim = sum(q.shape[1] for q in pieces)
        Wz, Wr, b = _layer_weights(p, in_dim)
        Z, R = _matmul_zr(pieces + [Phi], Wz, Wr, b)
        return _edge_op(Z, R, seg, inv, gidx, n)

    h = run([x], params['input'])
    feats = None
    for blk in params['blocks']:
        feats = [h]
        for lyr in blk['layers']:
            t = run(list(feats), lyr['conv1'])
            t = run([t], lyr['conv2'])
            feats.append(t)
        h = run(list(feats), blk['trans'])
    return _cls_readout([h], params['W_cls'], params['b_cls'], node2graph)


# speedup vs baseline: 7.3654x; 4.5782x over previous
import functools
from functools import partial

import jax
import jax.numpy as jnp
import numpy as np
from jax import lax
from jax.experimental import pallas as pl
from jax.experimental.pallas import tpu as pltpu
from jax.experimental.pallas import tpu_sc as plsc

N_NODES = 10000
N_EDGES = 320000
NUM_REL = 4
EDGE_DIM = 16
NUM_GRAPHS = 64
EPS = 1e-10
BN_SCALE = 1.0 / np.sqrt(1.0 + 1e-5)

ROW_BLK = 2000
TW = 32
NSEG = N_NODES * NUM_REL
ACC_ROWS = NSEG + 64
E_PAD = 2560 * 128
IDX_ROWS = E_PAD // 128
MAC_ROWS = 4


def _zr_body(n_pieces, n_tab, *refs):
    pieces = refs[:n_pieces]
    wz = refs[n_pieces:n_pieces + n_tab]
    wr_ref = refs[n_pieces + n_tab]
    b_ref = refs[n_pieces + n_tab + 1]
    zouts = refs[n_pieces + n_tab + 2: n_pieces + 2 * n_tab + 2]
    r_ref = refs[n_pieces + 2 * n_tab + 2]
    if n_pieces == 1:
        a = pieces[0][...]
    else:
        a = jnp.concatenate([p[...] for p in pieces], axis=1)
    for k in range(n_tab):
        zouts[k][...] = jnp.dot(a, wz[k][...], preferred_element_type=jnp.float32)
    r_ref[...] = jnp.dot(a, wr_ref[...], preferred_element_type=jnp.float32) + b_ref[...]


def _matmul_zr(pieces, Wz_list, Wr, b):
    n = pieces[0].shape[0]
    C = len(Wz_list)
    rout = Wr.shape[1]
    grid = n // ROW_BLK
    in_specs = [pl.BlockSpec((ROW_BLK, p.shape[1]), lambda i: (i, 0)) for p in pieces]
    in_specs += [pl.BlockSpec(w.shape, lambda i: (0, 0)) for w in Wz_list]
    in_specs += [
        pl.BlockSpec(Wr.shape, lambda i: (0, 0)),
        pl.BlockSpec((1, rout), lambda i: (0, 0)),
    ]
    out_specs = [pl.BlockSpec((ROW_BLK, NUM_REL * TW), lambda i: (i, 0)) for _ in range(C)]
    out_specs += [pl.BlockSpec((ROW_BLK, rout), lambda i: (i, 0))]
    outs = pl.pallas_call(
        partial(_zr_body, len(pieces), C),
        grid=(grid,),
        in_specs=in_specs,
        out_specs=out_specs,
        out_shape=[jax.ShapeDtypeStruct((n, NUM_REL * TW), jnp.float32) for _ in range(C)]
        + [jax.ShapeDtypeStruct((n, rout), jnp.float32)],
    )(*pieces, *Wz_list, Wr, b.reshape(1, rout))
    return outs[:C], outs[C]


@functools.lru_cache(maxsize=None)
def _sc_edge_kernel(C, split_last):
    full = list(range(C - 1 if split_last else C))
    n_out = len(full) + (2 if split_last else 0)
    mesh = plsc.VectorSubcoreMesh(core_axis_name="c", subcore_axis_name="s")
    out_type = [jax.ShapeDtypeStruct((NSEG, TW), jnp.float32) for _ in range(n_out)]
    scratch = [
        pltpu.VMEM((MAC_ROWS, 128), jnp.int32),
        pltpu.VMEM((MAC_ROWS, 128), jnp.int32),
        pltpu.VMEM((MAC_ROWS * 128, TW), jnp.float32),
        pltpu.VMEM_SHARED((ACC_ROWS, TW), jnp.float32),
        pltpu.SemaphoreType.DMA,
    ]

    @partial(pl.kernel, mesh=mesh, out_type=out_type, scratch_types=scratch,
             compiler_params=pltpu.CompilerParams(use_tc_tiling_on_sc=False))
    def body(gidx_ref, seg_ref, *rest):
        tabs = rest[:C]
        zeros_ref = rest[C]
        outs = rest[C + 1: C + 1 + n_out]
        gidx_v, seg_v, rows_v, acc, sem = rest[C + 1 + n_out:]
        c = lax.axis_index("c")
        t = lax.axis_index("s")
        zrows = ACC_ROWS // 16

        def run_pass(tab_ref, out_ref, row0, rpt):
            nmac = rpt // MAC_ROWS
            pltpu.sync_copy(zeros_ref.at[pl.ds(t * zrows, zrows)],
                            acc.at[pl.ds(t * zrows, zrows)])
            plsc.subcore_barrier()

            def macro(m, carry):
                base = row0 + t * rpt + m * MAC_ROWS
                pltpu.sync_copy(gidx_ref.at[pl.ds(base, MAC_ROWS)], gidx_v)
                pltpu.sync_copy(seg_ref.at[pl.ds(base, MAC_ROWS)], seg_v)
                cps = [pltpu.async_copy(tab_ref.at[gidx_v.at[j]],
                                        rows_v.at[pl.ds(j * 128, 128)], sem)
                       for j in range(MAC_ROWS)]
                for cp in cps:
                    cp.wait()
                for j in range(MAC_ROWS):
                    pltpu.sync_copy(rows_v.at[pl.ds(j * 128, 128)],
                                    acc.at[seg_v.at[j]], add=True)
                return carry

            lax.fori_loop(0, nmac, macro, 0)
            plsc.subcore_barrier()
            wrows = NSEG // 8

            @pl.when(t < 8)
            def _():
                pltpu.sync_copy(acc.at[pl.ds(t * wrows, wrows)],
                                out_ref.at[pl.ds(t * wrows, wrows)])

            plsc.subcore_barrier()

        @pl.when(c == 0)
        def _():
            for k in full[0::2]:
                run_pass(tabs[k], outs[k], 0, IDX_ROWS // 16)
            if split_last:
                run_pass(tabs[C - 1], outs[len(full)], 0, IDX_ROWS // 32)

        @pl.when(c == 1)
        def _():
            for k in full[1::2]:
                run_pass(tabs[k], outs[k], 0, IDX_ROWS // 16)
            if split_last:
                run_pass(tabs[C - 1], outs[len(full) + 1],
                         IDX_ROWS // 2, IDX_ROWS // 32)

    return body


def _epi_body(C, split_last, out_dim, *refs):
    n_acc = C + (1 if split_last else 0)
    accs = refs[:n_acc]
    inv_ref = refs[n_acc]
    r_ref = refs[n_acc + 1]
    x_ref = refs[n_acc + 2]
    inv = inv_ref[...]
    parts = []
    for k in range(C):
        A = accs[k][...]
        if split_last and k == C - 1:
            A = A + accs[C][...]
        w_real = min(TW, out_dim - k * TW)
        M = inv[:, 0:1] * A[:, 0 * TW:0 * TW + w_real]
        for r in range(1, NUM_REL):
            M = M + inv[:, r:r + 1] * A[:, r * TW:r * TW + w_real]
        parts.append(M)
    M = parts[0] if len(parts) == 1 else jnp.concatenate(parts, axis=1)
    x_ref[...] = jnp.maximum(M + r_ref[...], 0.0)


def _epilogue(accs, inv2, R, out_dim, C, split_last):
    n = N_NODES
    grid = n // ROW_BLK
    accs = [a.reshape(n, NUM_REL * TW) for a in accs]
    in_specs = [pl.BlockSpec((ROW_BLK, NUM_REL * TW), lambda i: (i, 0)) for _ in accs]
    in_specs += [
        pl.BlockSpec((ROW_BLK, NUM_REL), lambda i: (i, 0)),
        pl.BlockSpec((ROW_BLK, out_dim), lambda i: (i, 0)),
    ]
    return pl.pallas_call(
        partial(_epi_body, C, split_last, out_dim),
        grid=(grid,),
        in_specs=in_specs,
        out_specs=pl.BlockSpec((ROW_BLK, out_dim), lambda i: (i, 0)),
        out_shape=jax.ShapeDtypeStruct((n, out_dim), jnp.float32),
    )(*accs, inv2, R)


def _cls_body(n_pieces, *refs):
    i = pl.program_id(0)
    piece_refs = refs[:n_pieces]
    w_ref, b_ref, n2g_ref, nf_ref, gs_ref, cnt_ref = refs[n_pieces:]
    if n_pieces == 1:
        a = piece_refs[0][...]
    else:
        a = jnp.concatenate([p[...] for p in piece_refs], axis=1)
    nf = jnp.dot(a, w_ref[...], preferred_element_type=jnp.float32) + b_ref[...]
    nf_ref[...] = nf
    n2g = n2g_ref[0, 0, :]
    onehot = (n2g[None, :] == jax.lax.broadcasted_iota(jnp.int32, (NUM_GRAPHS, ROW_BLK), 0)
              ).astype(jnp.float32)
    part_s = jnp.dot(onehot, nf, preferred_element_type=jnp.float32)
    part_c = jnp.sum(onehot, axis=1, keepdims=True)

    @pl.when(i == 0)
    def _():
        gs_ref[...] = jnp.zeros_like(gs_ref)
        cnt_ref[...] = jnp.zeros_like(cnt_ref)

    gs_ref[...] += part_s
    cnt_ref[...] += part_c


def _cls_readout(pieces, W, b, node2graph):
    n = pieces[0].shape[0]
    out = W.shape[1]
    grid = n // ROW_BLK
    in_specs = [pl.BlockSpec((ROW_BLK, p.shape[1]), lambda i: (i, 0)) for p in pieces]
    in_specs += [
        pl.BlockSpec(W.shape, lambda i: (0, 0)),
        pl.BlockSpec((1, out), lambda i: (0, 0)),
        pl.BlockSpec((1, 1, ROW_BLK), lambda i: (i, 0, 0)),
    ]
    out_specs = [
        pl.BlockSpec((ROW_BLK, out), lambda i: (i, 0)),
        pl.BlockSpec((NUM_GRAPHS, out), lambda i: (0, 0)),
        pl.BlockSpec((NUM_GRAPHS, 1), lambda i: (0, 0)),
    ]
    nf, gs, cnt = pl.pallas_call(
        partial(_cls_body, len(pieces)),
        grid=(grid,),
        in_specs=in_specs,
        out_specs=out_specs,
        out_shape=[
            jax.ShapeDtypeStruct((n, out), jnp.float32),
            jax.ShapeDtypeStruct((NUM_GRAPHS, out), jnp.float32),
            jax.ShapeDtypeStruct((NUM_GRAPHS, 1), jnp.float32),
        ],
    )(*pieces, W, b.reshape(1, out),
      node2graph.reshape(n // ROW_BLK, 1, ROW_BLK).astype(jnp.int32))
    gf = gs / jnp.maximum(cnt, 1.0)
    return gf, nf


def _precompute_structure(edge_index, edge_type, edge_attr, n):
    src = edge_index[0].astype(jnp.int32)
    dst = edge_index[1].astype(jnp.int32)
    et = edge_type.astype(jnp.int32)
    seg = dst * NUM_REL + et
    den = jax.ops.segment_sum(jnp.ones((N_EDGES,), jnp.float32), seg,
                              num_segments=NSEG)
    inv = 1.0 / (den + EPS)
    Ef = jax.ops.segment_sum(edge_attr, seg, num_segments=NSEG)
    F = Ef * inv[:, None]
    c = den * inv
    Phi = jnp.concatenate([F.reshape(n, NUM_REL * EDGE_DIM),
                           c.reshape(n, NUM_REL)], axis=1)
    gidx = src * NUM_REL + et
    pad = E_PAD - N_EDGES
    gidx2d = jnp.concatenate([gidx, jnp.zeros((pad,), jnp.int32)]).reshape(IDX_ROWS, 128)
    seg2d = jnp.concatenate([seg, jnp.full((pad,), NSEG, jnp.int32)]).reshape(IDX_ROWS, 128)
    inv2 = inv.reshape(n, NUM_REL)
    return seg2d, gidx2d, inv2, Phi


def _layer_weights(p, in_dim):
    out_dim = p['W_loop'].shape[1]
    g = p['gamma'] * BN_SCALE
    Wlin = p['W_lin'].reshape(NUM_REL, in_dim, out_dim) * g
    C = -(-out_dim // TW)
    Wz_list = []
    for k in range(C):
        w_real = min(TW, out_dim - k * TW)
        Wk = Wlin[:, :, k * TW:k * TW + w_real]
        if w_real < TW:
            Wk = jnp.pad(Wk, ((0, 0), (0, 0), (0, TW - w_real)))
        Wz = Wk.transpose(1, 0, 2).reshape(in_dim, NUM_REL * TW)
        Wz = jnp.concatenate([Wz, jnp.zeros((68, NUM_REL * TW), jnp.float32)], axis=0)
        Wz_list.append(Wz)
    U = jnp.einsum('ed,rdo->reo', p['W_edge'], Wlin)
    V = jnp.einsum('d,rdo->ro', p['b_edge'], Wlin)
    G = jnp.concatenate([U.reshape(NUM_REL * EDGE_DIM, out_dim), V], axis=0)
    Wr = jnp.concatenate([p['W_loop'] * g, G], axis=0)
    b = (p['b_lin'] + p['b_loop']) * g + p['beta']
    return Wz_list, Wr, b, out_dim


def kernel(x, edge_index, edge_type, edge_attr, node2graph, params):
    n = x.shape[0]
    seg2d, gidx2d, inv2, Phi = _precompute_structure(edge_index, edge_type, edge_attr, n)
    zeros = jnp.zeros((ACC_ROWS, TW), jnp.float32)

    def run(pieces, p):
        in_dim = sum(q.shape[1] for q in pieces)
        Wz_list, Wr, b, out_dim = _layer_weights(p, in_dim)
        C = len(Wz_list)
        split_last = (C % 2 == 1)
        Zs, R = _matmul_zr(pieces + [Phi], Wz_list, Wr, b)
        Ztabs = [z.reshape(n * NUM_REL, TW) for z in Zs]
        accs = _sc_edge_kernel(C, split_last)(gidx2d, seg2d, *Ztabs, zeros)
        if not isinstance(accs, (list, tuple)):
            accs = [accs]
        return _epilogue(list(accs), inv2, R, out_dim, C, split_last)

    h = run([x], params['input'])
    feats = None
    for blk in params['blocks']:
        feats = [h]
        for lyr in blk['layers']:
            t = run(list(feats), lyr['conv1'])
            t = run([t], lyr['conv2'])
            feats.append(t)
        h = run(list(feats), blk['trans'])
    return _cls_readout([h], params['W_cls'], params['b_cls'], node2graph)


# speedup vs baseline: 8.8524x; 1.2019x over previous
import functools
from functools import partial

import jax
import jax.numpy as jnp
import numpy as np
from jax import lax
from jax.experimental import pallas as pl
from jax.experimental.pallas import tpu as pltpu
from jax.experimental.pallas import tpu_sc as plsc

N_NODES = 10000
N_EDGES = 320000
NUM_REL = 4
EDGE_DIM = 16
NUM_GRAPHS = 64
EPS = 1e-10
BN_SCALE = 1.0 / np.sqrt(1.0 + 1e-5)

ROW_BLK = 2000
TW = 32
NSEG = N_NODES * NUM_REL
ACC_ROWS = NSEG + 64
E_PAD = 2560 * 128
IDX_ROWS = E_PAD // 128
MAC_ROWS = 4


def _zr_body(n_pieces, n_tab, *refs):
    pieces = refs[:n_pieces]
    wz = refs[n_pieces:n_pieces + n_tab]
    wr_ref = refs[n_pieces + n_tab]
    b_ref = refs[n_pieces + n_tab + 1]
    zouts = refs[n_pieces + n_tab + 2: n_pieces + 2 * n_tab + 2]
    r_ref = refs[n_pieces + 2 * n_tab + 2]
    if n_pieces == 1:
        a = pieces[0][...]
    else:
        a = jnp.concatenate([p[...] for p in pieces], axis=1)
    for k in range(n_tab):
        zouts[k][...] = jnp.dot(a, wz[k][...], preferred_element_type=jnp.float32)
    r_ref[...] = jnp.dot(a, wr_ref[...], preferred_element_type=jnp.float32) + b_ref[...]


def _matmul_zr(pieces, Wz_list, Wr, b):
    n = pieces[0].shape[0]
    C = len(Wz_list)
    rout = Wr.shape[1]
    grid = n // ROW_BLK
    in_specs = [pl.BlockSpec((ROW_BLK, p.shape[1]), lambda i: (i, 0)) for p in pieces]
    in_specs += [pl.BlockSpec(w.shape, lambda i: (0, 0)) for w in Wz_list]
    in_specs += [
        pl.BlockSpec(Wr.shape, lambda i: (0, 0)),
        pl.BlockSpec((1, rout), lambda i: (0, 0)),
    ]
    out_specs = [pl.BlockSpec((ROW_BLK, NUM_REL * TW), lambda i: (i, 0)) for _ in range(C)]
    out_specs += [pl.BlockSpec((ROW_BLK, rout), lambda i: (i, 0))]
    outs = pl.pallas_call(
        partial(_zr_body, len(pieces), C),
        grid=(grid,),
        in_specs=in_specs,
        out_specs=out_specs,
        out_shape=[jax.ShapeDtypeStruct((n, NUM_REL * TW), jnp.float32) for _ in range(C)]
        + [jax.ShapeDtypeStruct((n, rout), jnp.float32)],
    )(*pieces, *Wz_list, Wr, b.reshape(1, rout))
    return outs[:C], outs[C]


@functools.lru_cache(maxsize=None)
def _sc_edge_kernel(C, split_last):
    full = list(range(C - 1 if split_last else C))
    n_out = len(full) + (2 if split_last else 0)
    mesh = plsc.VectorSubcoreMesh(core_axis_name="c", subcore_axis_name="s")
    out_type = [jax.ShapeDtypeStruct((NSEG, TW), jnp.float32) for _ in range(n_out)]
    scratch = [
        pltpu.VMEM((2 * MAC_ROWS, 128), jnp.int32),
        pltpu.VMEM((2 * MAC_ROWS, 128), jnp.int32),
        pltpu.VMEM((2 * MAC_ROWS * 128, TW), jnp.float32),
        pltpu.VMEM_SHARED((ACC_ROWS, TW), jnp.float32),
        pltpu.SemaphoreType.DMA,
        pltpu.SemaphoreType.DMA,
    ]

    @partial(pl.kernel, mesh=mesh, out_type=out_type, scratch_types=scratch,
             compiler_params=pltpu.CompilerParams(use_tc_tiling_on_sc=False))
    def body(gidx_ref, seg_ref, *rest):
        tabs = rest[:C]
        zeros_ref = rest[C]
        outs = rest[C + 1: C + 1 + n_out]
        gidx_v, seg_v, rows_v, acc, sem_g, sem_s = rest[C + 1 + n_out:]
        c = lax.axis_index("c")
        t = lax.axis_index("s")
        zrows = ACC_ROWS // 16

        def run_pass(tab_ref, out_ref, row0, rpt):
            nmac = rpt // MAC_ROWS
            pltpu.sync_copy(zeros_ref.at[pl.ds(t * zrows, zrows)],
                            acc.at[pl.ds(t * zrows, zrows)])
            plsc.subcore_barrier()

            def row_slice(b, j):
                return rows_v.at[pl.ds((b * MAC_ROWS + j) * 128, 128)]

            def drain(b):
                for j in range(MAC_ROWS):
                    pltpu.make_async_copy(
                        row_slice(b, j),
                        acc.at[seg_v.at[b * MAC_ROWS + j]], sem_s).wait()

            def do_macro(b, m):
                base = row0 + t * rpt + m * MAC_ROWS
                pltpu.sync_copy(
                    gidx_ref.at[pl.ds(base, MAC_ROWS)],
                    gidx_v.at[pl.ds(b * MAC_ROWS, MAC_ROWS)])
                pltpu.sync_copy(
                    seg_ref.at[pl.ds(base, MAC_ROWS)],
                    seg_v.at[pl.ds(b * MAC_ROWS, MAC_ROWS)])
                cps = [pltpu.async_copy(tab_ref.at[gidx_v.at[b * MAC_ROWS + j]],
                                        row_slice(b, j), sem_g)
                       for j in range(MAC_ROWS)]
                for cp in cps:
                    cp.wait()
                for j in range(MAC_ROWS):
                    pltpu.async_copy(row_slice(b, j),
                                     acc.at[seg_v.at[b * MAC_ROWS + j]],
                                     sem_s, add=True)

            def pair(q, carry):
                for b in range(2):
                    @pl.when(q > 0)
                    def _():
                        drain(b)
                    do_macro(b, 2 * q + b)
                return carry

            lax.fori_loop(0, nmac // 2, pair, 0)
            drain(0)
            drain(1)
            plsc.subcore_barrier()
            wrows = NSEG // 8

            @pl.when(t < 8)
            def _():
                pltpu.sync_copy(acc.at[pl.ds(t * wrows, wrows)],
                                out_ref.at[pl.ds(t * wrows, wrows)])

            plsc.subcore_barrier()

        @pl.when(c == 0)
        def _():
            for k in full[0::2]:
                run_pass(tabs[k], outs[k], 0, IDX_ROWS // 16)
            if split_last:
                run_pass(tabs[C - 1], outs[len(full)], 0, IDX_ROWS // 32)

        @pl.when(c == 1)
        def _():
            for k in full[1::2]:
                run_pass(tabs[k], outs[k], 0, IDX_ROWS // 16)
            if split_last:
                run_pass(tabs[C - 1], outs[len(full) + 1],
                         IDX_ROWS // 2, IDX_ROWS // 32)

    return body


@functools.lru_cache(maxsize=None)
def _sc_pre_kernel():
    mesh = plsc.VectorSubcoreMesh(core_axis_name="c", subcore_axis_name="s")
    out_type = [jax.ShapeDtypeStruct((NSEG, TW), jnp.float32) for _ in range(2)]
    scratch = [
        pltpu.VMEM((2 * MAC_ROWS, 128), jnp.int32),
        pltpu.VMEM((2 * MAC_ROWS * 128, TW), jnp.float32),
        pltpu.VMEM_SHARED((ACC_ROWS, TW), jnp.float32),
        pltpu.SemaphoreType.DMA,
    ]

    @partial(pl.kernel, mesh=mesh, out_type=out_type, scratch_types=scratch,
             compiler_params=pltpu.CompilerParams(use_tc_tiling_on_sc=False))
    def body(seg_ref, ea_ref, zeros_ref, out0, out1, seg_v, rows_v, acc, sem_s):
        c = lax.axis_index("c")
        t = lax.axis_index("s")
        zrows = ACC_ROWS // 16
        rpt = IDX_ROWS // 32
        pltpu.sync_copy(zeros_ref.at[pl.ds(t * zrows, zrows)],
                        acc.at[pl.ds(t * zrows, zrows)])
        plsc.subcore_barrier()
        row0 = c * (IDX_ROWS // 2)

        def row_slice(b, j):
            return rows_v.at[pl.ds((b * MAC_ROWS + j) * 128, 128)]

        def drain(b):
            for j in range(MAC_ROWS):
                pltpu.make_async_copy(
                    row_slice(b, j),
                    acc.at[seg_v.at[b * MAC_ROWS + j]], sem_s).wait()

        def do_macro(b, m):
            base = row0 + t * rpt + m * MAC_ROWS
            pltpu.sync_copy(seg_ref.at[pl.ds(base, MAC_ROWS)],
                            seg_v.at[pl.ds(b * MAC_ROWS, MAC_ROWS)])
            pltpu.sync_copy(ea_ref.at[pl.ds(base * 128, MAC_ROWS * 128)],
                            rows_v.at[pl.ds(b * MAC_ROWS * 128, MAC_ROWS * 128)])
            for j in range(MAC_ROWS):
                pltpu.async_copy(row_slice(b, j),
                                 acc.at[seg_v.at[b * MAC_ROWS + j]],
                                 sem_s, add=True)

        def pair(q, carry):
            for b in range(2):
                @pl.when(q > 0)
                def _():
                    drain(b)
                do_macro(b, 2 * q + b)
            return carry

        lax.fori_loop(0, rpt // MAC_ROWS // 2, pair, 0)
        drain(0)
        drain(1)
        plsc.subcore_barrier()
        wrows = NSEG // 8

        @pl.when(t < 8)
        def _():
            @pl.when(c == 0)
            def _():
                pltpu.sync_copy(acc.at[pl.ds(t * wrows, wrows)],
                                out0.at[pl.ds(t * wrows, wrows)])

            @pl.when(c == 1)
            def _():
                pltpu.sync_copy(acc.at[pl.ds(t * wrows, wrows)],
                                out1.at[pl.ds(t * wrows, wrows)])

        plsc.subcore_barrier()

    return body


def _epi_body(C, split_last, out_dim, *refs):
    n_acc = C + (1 if split_last else 0)
    accs = refs[:n_acc]
    inv_ref = refs[n_acc]
    r_ref = refs[n_acc + 1]
    x_ref = refs[n_acc + 2]
    inv = inv_ref[...]
    parts = []
    for k in range(C):
        A = accs[k][...]
        if split_last and k == C - 1:
            A = A + accs[C][...]
        w_real = min(TW, out_dim - k * TW)
        M = inv[:, 0:1] * A[:, 0 * TW:0 * TW + w_real]
        for r in range(1, NUM_REL):
            M = M + inv[:, r:r + 1] * A[:, r * TW:r * TW + w_real]
        parts.append(M)
    M = parts[0] if len(parts) == 1 else jnp.concatenate(parts, axis=1)
    x_ref[...] = jnp.maximum(M + r_ref[...], 0.0)


def _epilogue(accs, inv2, R, out_dim, C, split_last):
    n = N_NODES
    grid = n // ROW_BLK
    accs = [a.reshape(n, NUM_REL * TW) for a in accs]
    in_specs = [pl.BlockSpec((ROW_BLK, NUM_REL * TW), lambda i: (i, 0)) for _ in accs]
    in_specs += [
        pl.BlockSpec((ROW_BLK, NUM_REL), lambda i: (i, 0)),
        pl.BlockSpec((ROW_BLK, out_dim), lambda i: (i, 0)),
    ]
    return pl.pallas_call(
        partial(_epi_body, C, split_last, out_dim),
        grid=(grid,),
        in_specs=in_specs,
        out_specs=pl.BlockSpec((ROW_BLK, out_dim), lambda i: (i, 0)),
        out_shape=jax.ShapeDtypeStruct((n, out_dim), jnp.float32),
    )(*accs, inv2, R)


def _cls_body(n_pieces, *refs):
    i = pl.program_id(0)
    piece_refs = refs[:n_pieces]
    w_ref, b_ref, n2g_ref, nf_ref, gs_ref, cnt_ref = refs[n_pieces:]
    if n_pieces == 1:
        a = piece_refs[0][...]
    else:
        a = jnp.concatenate([p[...] for p in piece_refs], axis=1)
    nf = jnp.dot(a, w_ref[...], preferred_element_type=jnp.float32) + b_ref[...]
    nf_ref[...] = nf
    n2g = n2g_ref[0, 0, :]
    onehot = (n2g[None, :] == jax.lax.broadcasted_iota(jnp.int32, (NUM_GRAPHS, ROW_BLK), 0)
              ).astype(jnp.float32)
    part_s = jnp.dot(onehot, nf, preferred_element_type=jnp.float32)
    part_c = jnp.sum(onehot, axis=1, keepdims=True)

    @pl.when(i == 0)
    def _():
        gs_ref[...] = jnp.zeros_like(gs_ref)
        cnt_ref[...] = jnp.zeros_like(cnt_ref)

    gs_ref[...] += part_s
    cnt_ref[...] += part_c


def _cls_readout(pieces, W, b, node2graph):
    n = pieces[0].shape[0]
    out = W.shape[1]
    grid = n // ROW_BLK
    in_specs = [pl.BlockSpec((ROW_BLK, p.shape[1]), lambda i: (i, 0)) for p in pieces]
    in_specs += [
        pl.BlockSpec(W.shape, lambda i: (0, 0)),
        pl.BlockSpec((1, out), lambda i: (0, 0)),
        pl.BlockSpec((1, 1, ROW_BLK), lambda i: (i, 0, 0)),
    ]
    out_specs = [
        pl.BlockSpec((ROW_BLK, out), lambda i: (i, 0)),
        pl.BlockSpec((NUM_GRAPHS, out), lambda i: (0, 0)),
        pl.BlockSpec((NUM_GRAPHS, 1), lambda i: (0, 0)),
    ]
    nf, gs, cnt = pl.pallas_call(
        partial(_cls_body, len(pieces)),
        grid=(grid,),
        in_specs=in_specs,
        out_specs=out_specs,
        out_shape=[
            jax.ShapeDtypeStruct((n, out), jnp.float32),
            jax.ShapeDtypeStruct((NUM_GRAPHS, out), jnp.float32),
            jax.ShapeDtypeStruct((NUM_GRAPHS, 1), jnp.float32),
        ],
    )(*pieces, W, b.reshape(1, out),
      node2graph.reshape(n // ROW_BLK, 1, ROW_BLK).astype(jnp.int32))
    gf = gs / jnp.maximum(cnt, 1.0)
    return gf, nf


def _precompute_structure(edge_index, edge_type, edge_attr, n, zeros):
    src = edge_index[0].astype(jnp.int32)
    dst = edge_index[1].astype(jnp.int32)
    et = edge_type.astype(jnp.int32)
    seg = dst * NUM_REL + et
    gidx = src * NUM_REL + et
    pad = E_PAD - N_EDGES
    gidx2d = jnp.concatenate([gidx, jnp.zeros((pad,), jnp.int32)]).reshape(IDX_ROWS, 128)
    seg2d = jnp.concatenate([seg, jnp.full((pad,), NSEG, jnp.int32)]).reshape(IDX_ROWS, 128)
    ea32 = jnp.pad(
        jnp.concatenate([edge_attr, jnp.ones((N_EDGES, 1), jnp.float32)], axis=1),
        ((0, pad), (0, TW - EDGE_DIM - 1)))
    p0, p1 = _sc_pre_kernel()(seg2d, ea32, zeros)
    S = p0 + p1
    Ef = S[:, :EDGE_DIM]
    den = S[:, EDGE_DIM]
    inv = 1.0 / (den + EPS)
    F = Ef * inv[:, None]
    c = den * inv
    Phi = jnp.concatenate([F.reshape(n, NUM_REL * EDGE_DIM),
                           c.reshape(n, NUM_REL)], axis=1)
    inv2 = inv.reshape(n, NUM_REL)
    return seg2d, gidx2d, inv2, Phi


def _layer_weights(p, in_dim):
    out_dim = p['W_loop'].shape[1]
    g = p['gamma'] * BN_SCALE
    Wlin = p['W_lin'].reshape(NUM_REL, in_dim, out_dim) * g
    C = -(-out_dim // TW)
    Wz_list = []
    for k in range(C):
        w_real = min(TW, out_dim - k * TW)
        Wk = Wlin[:, :, k * TW:k * TW + w_real]
        if w_real < TW:
            Wk = jnp.pad(Wk, ((0, 0), (0, 0), (0, TW - w_real)))
        Wz = Wk.transpose(1, 0, 2).reshape(in_dim, NUM_REL * TW)
        Wz = jnp.concatenate([Wz, jnp.zeros((68, NUM_REL * TW), jnp.float32)], axis=0)
        Wz_list.append(Wz)
    U = jnp.einsum('ed,rdo->reo', p['W_edge'], Wlin)
    V = jnp.einsum('d,rdo->ro', p['b_edge'], Wlin)
    G = jnp.concatenate([U.reshape(NUM_REL * EDGE_DIM, out_dim), V], axis=0)
    Wr = jnp.concatenate([p['W_loop'] * g, G], axis=0)
    b = (p['b_lin'] + p['b_loop']) * g + p['beta']
    return Wz_list, Wr, b, out_dim


def kernel(x, edge_index, edge_type, edge_attr, node2graph, params):
    n = x.shape[0]
    zeros = jnp.zeros((ACC_ROWS, TW), jnp.float32)
    seg2d, gidx2d, inv2, Phi = _precompute_structure(
        edge_index, edge_type, edge_attr, n, zeros)

    def run(pieces, p):
        in_dim = sum(q.shape[1] for q in pieces)
        Wz_list, Wr, b, out_dim = _layer_weights(p, in_dim)
        C = len(Wz_list)
        split_last = (C % 2 == 1)
        Zs, R = _matmul_zr(pieces + [Phi], Wz_list, Wr, b)
        Ztabs = [z.reshape(n * NUM_REL, TW) for z in Zs]
        accs = _sc_edge_kernel(C, split_last)(gidx2d, seg2d, *Ztabs, zeros)
        if not isinstance(accs, (list, tuple)):
            accs = [accs]
        return _epilogue(list(accs), inv2, R, out_dim, C, split_last)

    h = run([x], params['input'])
    feats = None
    for blk in params['blocks']:
        feats = [h]
        for lyr in blk['layers']:
            t = run(list(feats), lyr['conv1'])
            t = run([t], lyr['conv2'])
            feats.append(t)
        h = run(list(feats), blk['trans'])
    return _cls_readout([h], params['W_cls'], params['b_cls'], node2graph)


# speedup vs baseline: 9.6225x; 1.0870x over previous
import functools
from functools import partial

import jax
import jax.numpy as jnp
import numpy as np
from jax import lax
from jax.experimental import pallas as pl
from jax.experimental.pallas import tpu as pltpu
from jax.experimental.pallas import tpu_sc as plsc

N_NODES = 10000
N_EDGES = 320000
NUM_REL = 4
EDGE_DIM = 16
NUM_GRAPHS = 64
EPS = 1e-10
BN_SCALE = 1.0 / np.sqrt(1.0 + 1e-5)

ROW_BLK = 2000
TW = 32
NSEG = N_NODES * NUM_REL
ACC_ROWS = NSEG + 64
E_PAD = 2560 * 128
IDX_ROWS = E_PAD // 128
MAC_ROWS = 4


def _zr_body(n_pieces, n_tab, *refs):
    pieces = refs[:n_pieces]
    wz = refs[n_pieces:n_pieces + n_tab]
    wr_ref = refs[n_pieces + n_tab]
    b_ref = refs[n_pieces + n_tab + 1]
    zouts = refs[n_pieces + n_tab + 2: n_pieces + 2 * n_tab + 2]
    r_ref = refs[n_pieces + 2 * n_tab + 2]
    if n_pieces == 1:
        a = pieces[0][...]
    else:
        a = jnp.concatenate([p[...] for p in pieces], axis=1)
    for k in range(n_tab):
        zouts[k][...] = jnp.dot(a, wz[k][...], preferred_element_type=jnp.float32)
    r_ref[...] = jnp.dot(a, wr_ref[...], preferred_element_type=jnp.float32) + b_ref[...]


def _matmul_zr(pieces, Wz_list, Wr, b):
    n = pieces[0].shape[0]
    C = len(Wz_list)
    rout = Wr.shape[1]
    grid = n // ROW_BLK
    in_specs = [pl.BlockSpec((ROW_BLK, p.shape[1]), lambda i: (i, 0)) for p in pieces]
    in_specs += [pl.BlockSpec(w.shape, lambda i: (0, 0)) for w in Wz_list]
    in_specs += [
        pl.BlockSpec(Wr.shape, lambda i: (0, 0)),
        pl.BlockSpec((1, rout), lambda i: (0, 0)),
    ]
    out_specs = [pl.BlockSpec((ROW_BLK, NUM_REL * TW), lambda i: (i, 0)) for _ in range(C)]
    out_specs += [pl.BlockSpec((ROW_BLK, rout), lambda i: (i, 0))]
    outs = pl.pallas_call(
        partial(_zr_body, len(pieces), C),
        grid=(grid,),
        in_specs=in_specs,
        out_specs=out_specs,
        out_shape=[jax.ShapeDtypeStruct((n, NUM_REL * TW), jnp.float32) for _ in range(C)]
        + [jax.ShapeDtypeStruct((n, rout), jnp.float32)],
    )(*pieces, *Wz_list, Wr, b.reshape(1, rout))
    return outs[:C], outs[C]


@functools.lru_cache(maxsize=None)
def _sc_edge_kernel(C, split_last):
    full = list(range(C - 1 if split_last else C))
    n_out = len(full) + (2 if split_last else 0)
    mesh = plsc.VectorSubcoreMesh(core_axis_name="c", subcore_axis_name="s")
    out_type = [jax.ShapeDtypeStruct((NSEG, TW), jnp.float32) for _ in range(n_out)]
    scratch = [
        pltpu.VMEM((2 * MAC_ROWS, 128), jnp.int32),
        pltpu.VMEM((2 * MAC_ROWS, 128), jnp.int32),
        pltpu.VMEM((2 * MAC_ROWS * 128, TW), jnp.float32),
        pltpu.VMEM_SHARED((ACC_ROWS, TW), jnp.float32),
        pltpu.SemaphoreType.DMA,
        pltpu.SemaphoreType.DMA,
        pltpu.SemaphoreType.DMA,
        pltpu.SemaphoreType.DMA,
    ]

    @partial(pl.kernel, mesh=mesh, out_type=out_type, scratch_types=scratch,
             compiler_params=pltpu.CompilerParams(use_tc_tiling_on_sc=False))
    def body(gidx_ref, seg_ref, *rest):
        tabs = rest[:C]
        zeros_ref = rest[C]
        outs = rest[C + 1: C + 1 + n_out]
        gidx_v, seg_v, rows_v, acc, sg0, sg1, ss0, ss1 = rest[C + 1 + n_out:]
        sem_g = (sg0, sg1)
        sem_s = (ss0, ss1)
        c = lax.axis_index("c")
        t = lax.axis_index("s")
        zrows = ACC_ROWS // 16

        def run_pass(tab_ref, out_ref, row0, rpt):
            nmac = rpt // MAC_ROWS
            pltpu.sync_copy(zeros_ref.at[pl.ds(t * zrows, zrows)],
                            acc.at[pl.ds(t * zrows, zrows)])
            plsc.subcore_barrier()

            def row_slice(b, j):
                return rows_v.at[pl.ds((b * MAC_ROWS + j) * 128, 128)]

            def drain(b):
                for j in range(MAC_ROWS):
                    pltpu.make_async_copy(
                        row_slice(b, j),
                        acc.at[seg_v.at[b * MAC_ROWS + j]], sem_s[b]).wait()

            def load_and_fire(b, m):
                base = row0 + t * rpt + m * MAC_ROWS
                pltpu.sync_copy(
                    gidx_ref.at[pl.ds(base, MAC_ROWS)],
                    gidx_v.at[pl.ds(b * MAC_ROWS, MAC_ROWS)])
                pltpu.sync_copy(
                    seg_ref.at[pl.ds(base, MAC_ROWS)],
                    seg_v.at[pl.ds(b * MAC_ROWS, MAC_ROWS)])
                for j in range(MAC_ROWS):
                    pltpu.async_copy(tab_ref.at[gidx_v.at[b * MAC_ROWS + j]],
                                     row_slice(b, j), sem_g[b])

            def wait_and_scatter(b):
                for j in range(MAC_ROWS):
                    pltpu.make_async_copy(tab_ref.at[gidx_v.at[b * MAC_ROWS + j]],
                                          row_slice(b, j), sem_g[b]).wait()
                for j in range(MAC_ROWS):
                    pltpu.async_copy(row_slice(b, j),
                                     acc.at[seg_v.at[b * MAC_ROWS + j]],
                                     sem_s[b], add=True)

            def pair(q, carry):
                for b in range(2):
                    @pl.when(q > 0)
                    def _():
                        drain(b)
                    load_and_fire(b, 2 * q + b)
                for b in range(2):
                    wait_and_scatter(b)
                return carry

            lax.fori_loop(0, nmac // 2, pair, 0)
            drain(0)
            drain(1)
            plsc.subcore_barrier()
            wrows = NSEG // 8

            @pl.when(t < 8)
            def _():
                pltpu.sync_copy(acc.at[pl.ds(t * wrows, wrows)],
                                out_ref.at[pl.ds(t * wrows, wrows)])

            plsc.subcore_barrier()

        @pl.when(c == 0)
        def _():
            for k in full[0::2]:
                run_pass(tabs[k], outs[k], 0, IDX_ROWS // 16)
            if split_last:
                run_pass(tabs[C - 1], outs[len(full)], 0, IDX_ROWS // 32)

        @pl.when(c == 1)
        def _():
            for k in full[1::2]:
                run_pass(tabs[k], outs[k], 0, IDX_ROWS // 16)
            if split_last:
                run_pass(tabs[C - 1], outs[len(full) + 1],
                         IDX_ROWS // 2, IDX_ROWS // 32)

    return body


@functools.lru_cache(maxsize=None)
def _sc_pre_kernel():
    mesh = plsc.VectorSubcoreMesh(core_axis_name="c", subcore_axis_name="s")
    out_type = [jax.ShapeDtypeStruct((NSEG, TW), jnp.float32) for _ in range(2)]
    scratch = [
        pltpu.VMEM((2 * MAC_ROWS, 128), jnp.int32),
        pltpu.VMEM((2 * MAC_ROWS * 128, TW), jnp.float32),
        pltpu.VMEM_SHARED((ACC_ROWS, TW), jnp.float32),
        pltpu.SemaphoreType.DMA,
        pltpu.SemaphoreType.DMA,
    ]

    @partial(pl.kernel, mesh=mesh, out_type=out_type, scratch_types=scratch,
             compiler_params=pltpu.CompilerParams(use_tc_tiling_on_sc=False))
    def body(seg_ref, ea_ref, zeros_ref, out0, out1, seg_v, rows_v, acc, ps0, ps1):
        sem_s = (ps0, ps1)
        c = lax.axis_index("c")
        t = lax.axis_index("s")
        zrows = ACC_ROWS // 16
        rpt = IDX_ROWS // 32
        pltpu.sync_copy(zeros_ref.at[pl.ds(t * zrows, zrows)],
                        acc.at[pl.ds(t * zrows, zrows)])
        plsc.subcore_barrier()
        row0 = c * (IDX_ROWS // 2)

        def row_slice(b, j):
            return rows_v.at[pl.ds((b * MAC_ROWS + j) * 128, 128)]

        def drain(b):
            for j in range(MAC_ROWS):
                pltpu.make_async_copy(
                    row_slice(b, j),
                    acc.at[seg_v.at[b * MAC_ROWS + j]], sem_s[b]).wait()

        def do_macro(b, m):
            base = row0 + t * rpt + m * MAC_ROWS
            pltpu.sync_copy(seg_ref.at[pl.ds(base, MAC_ROWS)],
                            seg_v.at[pl.ds(b * MAC_ROWS, MAC_ROWS)])
            pltpu.sync_copy(ea_ref.at[pl.ds(base * 128, MAC_ROWS * 128)],
                            rows_v.at[pl.ds(b * MAC_ROWS * 128, MAC_ROWS * 128)])
            for j in range(MAC_ROWS):
                pltpu.async_copy(row_slice(b, j),
                                 acc.at[seg_v.at[b * MAC_ROWS + j]],
                                 sem_s[b], add=True)

        def pair(q, carry):
            for b in range(2):
                @pl.when(q > 0)
                def _():
                    drain(b)
                do_macro(b, 2 * q + b)
            return carry

        lax.fori_loop(0, rpt // MAC_ROWS // 2, pair, 0)
        drain(0)
        drain(1)
        plsc.subcore_barrier()
        wrows = NSEG // 8

        @pl.when(t < 8)
        def _():
            @pl.when(c == 0)
            def _():
                pltpu.sync_copy(acc.at[pl.ds(t * wrows, wrows)],
                                out0.at[pl.ds(t * wrows, wrows)])

            @pl.when(c == 1)
            def _():
                pltpu.sync_copy(acc.at[pl.ds(t * wrows, wrows)],
                                out1.at[pl.ds(t * wrows, wrows)])

        plsc.subcore_barrier()

    return body


def _epi_body(C, split_last, out_dim, *refs):
    n_acc = C + (1 if split_last else 0)
    accs = refs[:n_acc]
    inv_ref = refs[n_acc]
    r_ref = refs[n_acc + 1]
    x_ref = refs[n_acc + 2]
    inv = inv_ref[...]
    parts = []
    for k in range(C):
        A = accs[k][...]
        if split_last and k == C - 1:
            A = A + accs[C][...]
        w_real = min(TW, out_dim - k * TW)
        M = inv[:, 0:1] * A[:, 0 * TW:0 * TW + w_real]
        for r in range(1, NUM_REL):
            M = M + inv[:, r:r + 1] * A[:, r * TW:r * TW + w_real]
        parts.append(M)
    M = parts[0] if len(parts) == 1 else jnp.concatenate(parts, axis=1)
    x_ref[...] = jnp.maximum(M + r_ref[...], 0.0)


def _epilogue(accs, inv2, R, out_dim, C, split_last):
    n = N_NODES
    grid = n // ROW_BLK
    accs = [a.reshape(n, NUM_REL * TW) for a in accs]
    in_specs = [pl.BlockSpec((ROW_BLK, NUM_REL * TW), lambda i: (i, 0)) for _ in accs]
    in_specs += [
        pl.BlockSpec((ROW_BLK, NUM_REL), lambda i: (i, 0)),
        pl.BlockSpec((ROW_BLK, out_dim), lambda i: (i, 0)),
    ]
    return pl.pallas_call(
        partial(_epi_body, C, split_last, out_dim),
        grid=(grid,),
        in_specs=in_specs,
        out_specs=pl.BlockSpec((ROW_BLK, out_dim), lambda i: (i, 0)),
        out_shape=jax.ShapeDtypeStruct((n, out_dim), jnp.float32),
    )(*accs, inv2, R)


def _cls_body(n_pieces, *refs):
    i = pl.program_id(0)
    piece_refs = refs[:n_pieces]
    w_ref, b_ref, n2g_ref, nf_ref, gs_ref, cnt_ref = refs[n_pieces:]
    if n_pieces == 1:
        a = piece_refs[0][...]
    else:
        a = jnp.concatenate([p[...] for p in piece_refs], axis=1)
    nf = jnp.dot(a, w_ref[...], preferred_element_type=jnp.float32) + b_ref[...]
    nf_ref[...] = nf
    n2g = n2g_ref[0, 0, :]
    onehot = (n2g[None, :] == jax.lax.broadcasted_iota(jnp.int32, (NUM_GRAPHS, ROW_BLK), 0)
              ).astype(jnp.float32)
    part_s = jnp.dot(onehot, nf, preferred_element_type=jnp.float32)
    part_c = jnp.sum(onehot, axis=1, keepdims=True)

    @pl.when(i == 0)
    def _():
        gs_ref[...] = jnp.zeros_like(gs_ref)
        cnt_ref[...] = jnp.zeros_like(cnt_ref)

    gs_ref[...] += part_s
    cnt_ref[...] += part_c


def _cls_readout(pieces, W, b, node2graph):
    n = pieces[0].shape[0]
    out = W.shape[1]
    grid = n // ROW_BLK
    in_specs = [pl.BlockSpec((ROW_BLK, p.shape[1]), lambda i: (i, 0)) for p in pieces]
    in_specs += [
        pl.BlockSpec(W.shape, lambda i: (0, 0)),
        pl.BlockSpec((1, out), lambda i: (0, 0)),
        pl.BlockSpec((1, 1, ROW_BLK), lambda i: (i, 0, 0)),
    ]
    out_specs = [
        pl.BlockSpec((ROW_BLK, out), lambda i: (i, 0)),
        pl.BlockSpec((NUM_GRAPHS, out), lambda i: (0, 0)),
        pl.BlockSpec((NUM_GRAPHS, 1), lambda i: (0, 0)),
    ]
    nf, gs, cnt = pl.pallas_call(
        partial(_cls_body, len(pieces)),
        grid=(grid,),
        in_specs=in_specs,
        out_specs=out_specs,
        out_shape=[
            jax.ShapeDtypeStruct((n, out), jnp.float32),
            jax.ShapeDtypeStruct((NUM_GRAPHS, out), jnp.float32),
            jax.ShapeDtypeStruct((NUM_GRAPHS, 1), jnp.float32),
        ],
    )(*pieces, W, b.reshape(1, out),
      node2graph.reshape(n // ROW_BLK, 1, ROW_BLK).astype(jnp.int32))
    gf = gs / jnp.maximum(cnt, 1.0)
    return gf, nf


def _precompute_structure(edge_index, edge_type, edge_attr, n, zeros):
    src = edge_index[0].astype(jnp.int32)
    dst = edge_index[1].astype(jnp.int32)
    et = edge_type.astype(jnp.int32)
    seg = dst * NUM_REL + et
    gidx = src * NUM_REL + et
    pad = E_PAD - N_EDGES
    gidx2d = jnp.concatenate([gidx, jnp.zeros((pad,), jnp.int32)]).reshape(IDX_ROWS, 128)
    seg2d = jnp.concatenate([seg, jnp.full((pad,), NSEG, jnp.int32)]).reshape(IDX_ROWS, 128)
    ea32 = jnp.pad(
        jnp.concatenate([edge_attr, jnp.ones((N_EDGES, 1), jnp.float32)], axis=1),
        ((0, pad), (0, TW - EDGE_DIM - 1)))
    p0, p1 = _sc_pre_kernel()(seg2d, ea32, zeros)
    S = p0 + p1
    Ef = S[:, :EDGE_DIM]
    den = S[:, EDGE_DIM]
    inv = 1.0 / (den + EPS)
    F = Ef * inv[:, None]
    c = den * inv
    Phi = jnp.concatenate([F.reshape(n, NUM_REL * EDGE_DIM),
                           c.reshape(n, NUM_REL)], axis=1)
    inv2 = inv.reshape(n, NUM_REL)
    return seg2d, gidx2d, inv2, Phi


def _layer_weights(p, in_dim):
    out_dim = p['W_loop'].shape[1]
    g = p['gamma'] * BN_SCALE
    Wlin = p['W_lin'].reshape(NUM_REL, in_dim, out_dim) * g
    C = -(-out_dim // TW)
    Wz_list = []
    for k in range(C):
        w_real = min(TW, out_dim - k * TW)
        Wk = Wlin[:, :, k * TW:k * TW + w_real]
        if w_real < TW:
            Wk = jnp.pad(Wk, ((0, 0), (0, 0), (0, TW - w_real)))
        Wz = Wk.transpose(1, 0, 2).reshape(in_dim, NUM_REL * TW)
        Wz = jnp.concatenate([Wz, jnp.zeros((68, NUM_REL * TW), jnp.float32)], axis=0)
        Wz_list.append(Wz)
    U = jnp.einsum('ed,rdo->reo', p['W_edge'], Wlin)
    V = jnp.einsum('d,rdo->ro', p['b_edge'], Wlin)
    G = jnp.concatenate([U.reshape(NUM_REL * EDGE_DIM, out_dim), V], axis=0)
    Wr = jnp.concatenate([p['W_loop'] * g, G], axis=0)
    b = (p['b_lin'] + p['b_loop']) * g + p['beta']
    return Wz_list, Wr, b, out_dim


def kernel(x, edge_index, edge_type, edge_attr, node2graph, params):
    n = x.shape[0]
    zeros = jnp.zeros((ACC_ROWS, TW), jnp.float32)
    seg2d, gidx2d, inv2, Phi = _precompute_structure(
        edge_index, edge_type, edge_attr, n, zeros)

    def run(pieces, p):
        in_dim = sum(q.shape[1] for q in pieces)
        Wz_list, Wr, b, out_dim = _layer_weights(p, in_dim)
        C = len(Wz_list)
        split_last = (C % 2 == 1)
        Zs, R = _matmul_zr(pieces + [Phi], Wz_list, Wr, b)
        Ztabs = [z.reshape(n * NUM_REL, TW) for z in Zs]
        accs = _sc_edge_kernel(C, split_last)(gidx2d, seg2d, *Ztabs, zeros)
        if not isinstance(accs, (list, tuple)):
            accs = [accs]
        return _epilogue(list(accs), inv2, R, out_dim, C, split_last)

    h = run([x], params['input'])
    feats = None
    for blk in params['blocks']:
        feats = [h]
        for lyr in blk['layers']:
            t = run(list(feats), lyr['conv1'])
            t = run([t], lyr['conv2'])
            feats.append(t)
        h = run(list(feats), blk['trans'])
    return _cls_readout([h], params['W_cls'], params['b_cls'], node2graph)
